# Initial kernel scaffold; baseline (speedup 1.0000x reference)
import jax
import jax.numpy as jnp
from jax.experimental import pallas as pl


def kernel(edge_index, edge_weight, X):
    raise NotImplementedError("write your pallas kernel here")


# speedup vs baseline: 3.1141x; 3.1141x over previous
import functools

import jax
import jax.numpy as jnp
from jax import lax
from jax.experimental import pallas as pl
from jax.experimental.pallas import tpu as pltpu
from jax.experimental.pallas import tpu_sc as plsc

N_NODES = 10000
D = 128
DH = D // 2
NC = 2
NS = 16
L = 16
CHUNK = 128

ROWS_PER_TILE = N_NODES // NS
ZROWS = 125


def _sc_body(n_chunks, rows_hbm, cols_hbm, w_hbm, x_hbm, out_hbm,
             idx_c, idx_r, wv, gb, zb, shared, gsem):
    c = lax.axis_index("c")
    s = lax.axis_index("s")

    if True:
        zeros = jnp.zeros((L,), jnp.float32)
        def zrow(i, _):
            for v in range(DH // L):
                zb[i, pl.ds(v * L, L)] = zeros
            return 0
        lax.fori_loop(0, ZROWS, zrow, 0)
        for k in range(ROWS_PER_TILE // ZROWS):
            pltpu.sync_copy(zb, shared.at[pl.ds(s * ROWS_PER_TILE + k * ZROWS, ZROWS), :])
        plsc.subcore_barrier()

        col_off = c * N_NODES

        def chunk_body(j, _):
            base = (s * n_chunks + j) * CHUNK
            pltpu.sync_copy(cols_hbm.at[pl.ds(base, CHUNK)], idx_c.at[0])
            pltpu.sync_copy(rows_hbm.at[pl.ds(base, CHUNK)], idx_r.at[0])
            pltpu.sync_copy(w_hbm.at[pl.ds(base, CHUNK)], wv.at[0])
            for v in range(CHUNK // L):
                idx_c[0, pl.ds(v * L, L)] = idx_c[0, pl.ds(v * L, L)] + col_off
            pltpu.async_copy(x_hbm.at[idx_c.at[0]], gb.at[0], gsem).wait()

            zero_idx = jnp.zeros((L,), jnp.int32)

            def escale(e, _):
                w16 = plsc.load_gather(wv, [zero_idx, jnp.broadcast_to(e, (L,))])
                for v in range(DH // L):
                    gb[0, e, pl.ds(v * L, L)] = gb[0, e, pl.ds(v * L, L)] * w16
                return 0
            lax.fori_loop(0, CHUNK, escale, 0)
            pltpu.sync_copy(gb.at[0], shared.at[idx_r.at[0]], add=True)
            return 0
        lax.fori_loop(0, n_chunks, chunk_body, 0)
        plsc.subcore_barrier()

        for k in range(ROWS_PER_TILE // ZROWS):
            r0 = s * ROWS_PER_TILE + k * ZROWS
            pltpu.sync_copy(shared.at[pl.ds(r0, ZROWS), :],
                            out_hbm.at[pl.ds(r0, ZROWS), pl.ds(c * DH, DH)])


def kernel(edge_index, edge_weight, X):
    rows = edge_index[0].astype(jnp.int32)
    cols = edge_index[1].astype(jnp.int32)
    w = edge_weight.astype(jnp.float32)
    n_edges = rows.shape[0]

    epc = NS * CHUNK
    n_pad = (-n_edges) % epc
    if n_pad:
        rows = jnp.concatenate([rows, jnp.zeros((n_pad,), jnp.int32)])
        cols = jnp.concatenate([cols, jnp.zeros((n_pad,), jnp.int32)])
        w = jnp.concatenate([w, jnp.zeros((n_pad,), jnp.float32)])
    n_chunks = (n_edges + n_pad) // epc

    x2 = jnp.concatenate([X[:, :DH], X[:, DH:]], axis=0)

    mesh = plsc.VectorSubcoreMesh(core_axis_name="c", subcore_axis_name="s")
    f = pl.kernel(
        functools.partial(_sc_body, n_chunks),
        out_type=jax.ShapeDtypeStruct((N_NODES, D), jnp.float32),
        mesh=mesh,
        compiler_params=pltpu.CompilerParams(use_tc_tiling_on_sc=False,
                                             needs_layout_passes=False),
        scratch_types=[
            pltpu.VMEM((2, CHUNK), jnp.int32),
            pltpu.VMEM((2, CHUNK), jnp.int32),
            pltpu.VMEM((2, CHUNK), jnp.float32),
            pltpu.VMEM((2, CHUNK, DH), jnp.float32),
            pltpu.VMEM((ZROWS, DH), jnp.float32),
            pltpu.VMEM_SHARED((N_NODES, DH), jnp.float32),
            pltpu.SemaphoreType.DMA,
        ],
    )
    return f(rows, cols, w, x2)


# speedup vs baseline: 5.5077x; 1.7687x over previous
import functools

import jax
import jax.numpy as jnp
from jax import lax
from jax.experimental import pallas as pl
from jax.experimental.pallas import tpu as pltpu
from jax.experimental.pallas import tpu_sc as plsc

N_NODES = 10000
D = 128
DH = D // 2
NC = 2
NS = 16
L = 16
CHUNK = 128
NBUF = 3

ROWS_PER_TILE = N_NODES // NS
ZROWS = 125


def _sc_body(n_chunks, rows_hbm, cols_hbm, w_hbm, x_hbm, out_hbm,
             idxc, idxr, wv, gb, shared, gsem, ssem):
    c = lax.axis_index("c")
    s = lax.axis_index("s")

    pltpu.sync_copy(cols_hbm.at[c, s], idxc)
    pltpu.sync_copy(rows_hbm.at[s], idxr)
    pltpu.sync_copy(w_hbm.at[s], wv)

    zeros = jnp.zeros((L,), jnp.float32)

    def zrow(i, _):
        for v in range(DH // L):
            gb[0, i, pl.ds(v * L, L)] = zeros
        return 0
    lax.fori_loop(0, CHUNK, zrow, 0)
    for k in range(ROWS_PER_TILE // ZROWS):
        pltpu.sync_copy(gb.at[0, pl.ds(0, ZROWS), :],
                        shared.at[pl.ds(s * ROWS_PER_TILE + k * ZROWS, ZROWS), :])
    plsc.subcore_barrier()

    def gather_desc(j, b):
        return pltpu.make_async_copy(
            x_hbm.at[idxc.at[pl.ds(j * CHUNK, CHUNK)]], gb.at[b], gsem)

    def scatter_desc(j, b):
        return pltpu.make_async_copy(gb.at[b], shared.at[idxr.at[j]], ssem)

    for b in range(NBUF - 1):
        gather_desc(b, b).start()

    def outer(jo, _):
        for b in range(NBUF):
            j = jo * NBUF + b
            gather_desc(j, b).wait()
            pb = (b + NBUF - 1) % NBUF

            @pl.when(j >= 1)
            def _():
                scatter_desc(j - 1, pb).wait()

            @pl.when(j + NBUF - 1 < n_chunks)
            def _():
                gather_desc(j + NBUF - 1, pb).start()

            jbase = j * CHUNK

            def escale(e, _):
                w16 = plsc.load_gather(wv, [jnp.broadcast_to(jbase + e, (L,))])
                for v in range(DH // L):
                    gb[b, e, pl.ds(v * L, L)] = gb[b, e, pl.ds(v * L, L)] * w16
                return 0
            lax.fori_loop(0, CHUNK, escale, 0, unroll=2)
            pltpu.async_copy(gb.at[b], shared.at[idxr.at[j]], ssem, add=True)
        return 0
    lax.fori_loop(0, n_chunks // NBUF, outer, 0)
    scatter_desc(n_chunks - 1, (n_chunks - 1) % NBUF).wait()
    plsc.subcore_barrier()

    for k in range(ROWS_PER_TILE // ZROWS):
        r0 = s * ROWS_PER_TILE + k * ZROWS
        pltpu.sync_copy(shared.at[pl.ds(r0, ZROWS), :],
                        out_hbm.at[pl.ds(r0, ZROWS), pl.ds(c * DH, DH)])


def kernel(edge_index, edge_weight, X):
    rows = edge_index[0].astype(jnp.int32)
    cols = edge_index[1].astype(jnp.int32)
    w = edge_weight.astype(jnp.float32)
    n_edges = rows.shape[0]

    epg = NS * CHUNK * NBUF
    n_pad = (-n_edges) % epg
    if n_pad:
        rows = jnp.concatenate([rows, jnp.zeros((n_pad,), jnp.int32)])
        cols = jnp.concatenate([cols, jnp.zeros((n_pad,), jnp.int32)])
        w = jnp.concatenate([w, jnp.zeros((n_pad,), jnp.float32)])
    e_pad = n_edges + n_pad
    n_chunks = e_pad // (NS * CHUNK)
    per_tile = n_chunks * CHUNK

    cols2 = jnp.stack([cols, cols + N_NODES]).reshape(NC, NS, per_tile)
    rows3 = rows.reshape(NS, n_chunks, CHUNK)
    w2 = w.reshape(NS, per_tile)
    x2 = jnp.concatenate([X[:, :DH], X[:, DH:]], axis=0)

    mesh = plsc.VectorSubcoreMesh(core_axis_name="c", subcore_axis_name="s")
    f = pl.kernel(
        functools.partial(_sc_body, n_chunks),
        out_type=jax.ShapeDtypeStruct((N_NODES, D), jnp.float32),
        mesh=mesh,
        compiler_params=pltpu.CompilerParams(use_tc_tiling_on_sc=False,
                                             needs_layout_passes=False),
        scratch_types=[
            pltpu.VMEM((per_tile,), jnp.int32),
            pltpu.VMEM((n_chunks, CHUNK), jnp.int32),
            pltpu.VMEM((per_tile,), jnp.float32),
            pltpu.VMEM((NBUF, CHUNK, DH), jnp.float32),
            pltpu.VMEM_SHARED((N_NODES, DH), jnp.float32),
            pltpu.SemaphoreType.DMA,
            pltpu.SemaphoreType.DMA,
        ],
    )
    return f(rows3, cols2, w2, x2)


# speedup vs baseline: 6.0701x; 1.1021x over previous
import functools

import jax
import jax.numpy as jnp
from jax import lax
from jax.experimental import pallas as pl
from jax.experimental.pallas import tpu as pltpu
from jax.experimental.pallas import tpu_sc as plsc

N_NODES = 10000
D = 128
DH = D // 2
NC = 2
NS = 16
L = 16
CHUNK = 128
NBUF = 3

ROWS_PER_TILE = N_NODES // NS
ZROWS = 125


def _sc_body(n_chunks, rows_hbm, cols_hbm, w_hbm, x_hbm, out_hbm,
             idxc, idxr, wv, gb, shared, gsem, ssem):
    c = lax.axis_index("c")
    s = lax.axis_index("s")

    pltpu.sync_copy(cols_hbm.at[c, s], idxc)
    pltpu.sync_copy(rows_hbm.at[s], idxr)
    pltpu.sync_copy(w_hbm.at[s], wv)

    zeros = jnp.zeros((L,), jnp.float32)

    def zrow(i, _):
        for v in range(DH // L):
            gb[0, i, pl.ds(v * L, L)] = zeros
        return 0
    lax.fori_loop(0, CHUNK, zrow, 0)
    for k in range(ROWS_PER_TILE // ZROWS):
        pltpu.sync_copy(gb.at[0, pl.ds(0, ZROWS), :],
                        shared.at[pl.ds(s * ROWS_PER_TILE + k * ZROWS, ZROWS), :])
    plsc.subcore_barrier()

    def gather_desc(j, b):
        return pltpu.make_async_copy(
            x_hbm.at[idxc.at[pl.ds(j * CHUNK, CHUNK)]], gb.at[b], gsem)

    def scatter_desc(j, b):
        return pltpu.make_async_copy(gb.at[b], shared.at[idxr.at[j]], ssem)

    for b in range(NBUF - 1):
        gather_desc(b, b).start()

    def outer(jo, _):
        for b in range(NBUF):
            j = jo * NBUF + b
            gather_desc(j, b).wait()
            pb = (b + NBUF - 1) % NBUF
            jbase = j * CHUNK

            def escale(e, _):
                w16 = plsc.load_gather(wv, [jnp.broadcast_to(jbase + e, (L,))])
                for v in range(DH // L):
                    gb[b, e, pl.ds(v * L, L)] = gb[b, e, pl.ds(v * L, L)] * w16
                return 0
            lax.fori_loop(0, CHUNK, escale, 0, unroll=2)

            @pl.when(j >= 1)
            def _():
                scatter_desc(j - 1, pb).wait()

            @pl.when(j + NBUF - 1 < n_chunks)
            def _():
                gather_desc(j + NBUF - 1, pb).start()
            pltpu.async_copy(gb.at[b], shared.at[idxr.at[j]], ssem, add=True)
        return 0
    lax.fori_loop(0, n_chunks // NBUF, outer, 0)
    scatter_desc(n_chunks - 1, (n_chunks - 1) % NBUF).wait()
    plsc.subcore_barrier()

    for k in range(ROWS_PER_TILE // ZROWS):
        r0 = s * ROWS_PER_TILE + k * ZROWS
        pltpu.sync_copy(shared.at[pl.ds(r0, ZROWS), :],
                        out_hbm.at[pl.ds(r0, ZROWS), pl.ds(c * DH, DH)])


def kernel(edge_index, edge_weight, X):
    rows = edge_index[0].astype(jnp.int32)
    cols = edge_index[1].astype(jnp.int32)
    w = edge_weight.astype(jnp.float32)
    n_edges = rows.shape[0]

    epg = NS * CHUNK * NBUF
    n_pad = (-n_edges) % epg
    if n_pad:
        rows = jnp.concatenate([rows, jnp.zeros((n_pad,), jnp.int32)])
        cols = jnp.concatenate([cols, jnp.zeros((n_pad,), jnp.int32)])
        w = jnp.concatenate([w, jnp.zeros((n_pad,), jnp.float32)])
    e_pad = n_edges + n_pad
    n_chunks = e_pad // (NS * CHUNK)
    per_tile = n_chunks * CHUNK

    cols2 = jnp.stack([cols, cols + N_NODES]).reshape(NC, NS, per_tile)
    rows3 = rows.reshape(NS, n_chunks, CHUNK)
    w2 = w.reshape(NS, per_tile)
    x2 = jnp.concatenate([X[:, :DH], X[:, DH:]], axis=0)

    mesh = plsc.VectorSubcoreMesh(core_axis_name="c", subcore_axis_name="s")
    f = pl.kernel(
        functools.partial(_sc_body, n_chunks),
        out_type=jax.ShapeDtypeStruct((N_NODES, D), jnp.float32),
        mesh=mesh,
        compiler_params=pltpu.CompilerParams(use_tc_tiling_on_sc=False,
                                             needs_layout_passes=False),
        scratch_types=[
            pltpu.VMEM((per_tile,), jnp.int32),
            pltpu.VMEM((n_chunks, CHUNK), jnp.int32),
            pltpu.VMEM((per_tile,), jnp.float32),
            pltpu.VMEM((NBUF, CHUNK, DH), jnp.float32),
            pltpu.VMEM_SHARED((N_NODES, DH), jnp.float32),
            pltpu.SemaphoreType.DMA,
            pltpu.SemaphoreType.DMA,
        ],
    )
    return f(rows3, cols2, w2, x2)
